# Initial kernel scaffold; baseline (speedup 1.0000x reference)
import jax
import jax.numpy as jnp
from jax.experimental import pallas as pl


def kernel(x, edge_index, edge_attr):
    raise NotImplementedError("write your pallas kernel here")


# speedup vs baseline: 5.0025x; 5.0025x over previous
import functools
import jax
import jax.numpy as jnp
from jax import lax
from jax.experimental import pallas as pl
from jax.experimental.pallas import tpu as pltpu
from jax.experimental.pallas import tpu_sc as plsc

NC = 2
NS = 16
NW = NC * NS
L = 16
CHUNK = 128
K_HOPS = 3


def _full16(v):
    return jnp.full((L,), v, dtype=jnp.int32)


def _eas_body(ea_ref, out_ref):
    sel = (lax.broadcasted_iota(jnp.int32, (128, 32), 0) // 4
           == lax.broadcasted_iota(jnp.int32, (128, 32), 1)).astype(jnp.float32)
    out_ref[...] = jnp.dot(ea_ref[...], sel, preferred_element_type=jnp.float32)


def _dis_body(dp_ref, dis_ref):
    rows = dp_ref.shape[0] // 2
    deg = dp_ref[:rows, :] + dp_ref[rows:, :]
    safe = jnp.where(deg > 0, deg, 1.0)
    dis_ref[...] = jnp.where(deg > 0, lax.rsqrt(safe), 0.0)


def _combine_body(n, p_ref, o_ref):
    n_pad = p_ref.shape[0] // 2
    o_ref[...] = p_ref[:n, :] + p_ref[n_pad:n_pad + n, :]


def _deg_kernel(n_pad, npt, nchunks, col_hbm, val_hbm, part_hbm,
                col_v, val_v, zero_v, acc_sh):
    cid = lax.axis_index("c")
    sid = lax.axis_index("s")
    for i in range(npt // L):
        zero_v[pl.ds(i * L, L)] = jnp.zeros((L,), jnp.float32)
    pltpu.sync_copy(zero_v, acc_sh.at[pl.ds(sid * npt, npt)])
    plsc.subcore_barrier()

    wid = sid * NC + cid
    pltpu.sync_copy(col_hbm.at[wid], col_v)
    pltpu.sync_copy(val_hbm.at[wid], val_v)

    def body(j, _):
        pltpu.sync_copy(val_v.at[j], acc_sh.at[col_v.at[j]], add=True)
        return ()
    lax.fori_loop(0, nchunks, body, (), unroll=False)

    plsc.subcore_barrier()
    pltpu.sync_copy(acc_sh.at[pl.ds(sid * npt, npt)],
                    part_hbm.at[cid, pl.ds(sid * npt, npt)])


def _w_kernel(nchunks, row_hbm, col_hbm, eas_hbm, dis_hbm, w_hbm,
              row_v, col_v, eas_v, dis_v, w_v):
    cid = lax.axis_index("c")
    sid = lax.axis_index("s")
    wid = sid * NC + cid
    pltpu.sync_copy(dis_hbm, dis_v)
    pltpu.sync_copy(row_hbm.at[wid], row_v)
    pltpu.sync_copy(col_hbm.at[wid], col_v)
    pltpu.sync_copy(eas_hbm.at[wid], eas_v)

    def body(j, _):
        for g in range(CHUNK // L):
            sl = pl.ds(g * L, L)
            r16 = row_v[j, sl]
            c16 = col_v[j, sl]
            dr = plsc.load_gather(dis_v, [r16])
            dc = plsc.load_gather(dis_v, [c16])
            w_v[j, sl] = dr * eas_v[j, sl] * dc
        return ()
    lax.fori_loop(0, nchunks, body, (), unroll=False)
    pltpu.sync_copy(w_v, w_hbm.at[wid])


def kernel(x, edge_index, edge_attr):
    n, d = x.shape
    e = edge_index.shape[1]
    row = edge_index[0]
    col = edge_index[1]

    epw = ((e + NW * CHUNK - 1) // (NW * CHUNK)) * CHUNK
    e_pad = epw * NW
    nchunks = epw // CHUNK
    npt = ((n + NS * L - 1) // (NS * L)) * L
    n_pad = npt * NS

    row_p = jnp.pad(row, (0, e_pad - e)).reshape(NW, nchunks, CHUNK)
    col_p = jnp.pad(col, (0, e_pad - e)).reshape(NW, nchunks, CHUNK)
    ea_p = jnp.pad(edge_attr, ((0, e_pad - e), (0, 0)))

    eas = pl.pallas_call(
        _eas_body,
        out_shape=jax.ShapeDtypeStruct((e_pad // 32, 32), jnp.float32),
    )(ea_p.reshape(e_pad // 32, 128))
    eas_w = eas.reshape(NW, nchunks, CHUNK)

    deg_part = pl.kernel(
        functools.partial(_deg_kernel, n_pad, npt, nchunks),
        out_type=jax.ShapeDtypeStruct((NC, n_pad), jnp.float32),
        mesh=plsc.VectorSubcoreMesh(core_axis_name="c", subcore_axis_name="s", num_cores=NC, num_subcores=NS),
        compiler_params=pltpu.CompilerParams(needs_layout_passes=False),
        scratch_types=[
            pltpu.VMEM((nchunks, CHUNK), jnp.int32),
            pltpu.VMEM((nchunks, CHUNK), jnp.float32),
            pltpu.VMEM((npt,), jnp.float32),
            pltpu.VMEM_SHARED((n_pad,), jnp.float32),
        ],
    )(col_p, eas_w)

    dis = pl.pallas_call(
        _dis_body,
        out_shape=jax.ShapeDtypeStruct((n_pad // 128, 128), jnp.float32),
    )(deg_part.reshape(2 * (n_pad // 128), 128)).reshape(n_pad)

    w = pl.kernel(
        functools.partial(_w_kernel, nchunks),
        out_type=jax.ShapeDtypeStruct((NW, nchunks, CHUNK), jnp.float32),
        mesh=plsc.VectorSubcoreMesh(core_axis_name="c", subcore_axis_name="s", num_cores=NC, num_subcores=NS),
        compiler_params=pltpu.CompilerParams(needs_layout_passes=False),
        scratch_types=[
            pltpu.VMEM((nchunks, CHUNK), jnp.int32),
            pltpu.VMEM((nchunks, CHUNK), jnp.int32),
            pltpu.VMEM((nchunks, CHUNK), jnp.float32),
            pltpu.VMEM((n_pad,), jnp.float32),
            pltpu.VMEM((nchunks, CHUNK), jnp.float32),
        ],
    )(row_p, col_p, eas_w, dis)

    def _hop_body(x_hbm, row_hbm, col_hbm, w_hbm, part_hbm,
                  row_v, col_v, w_v, rows_v, sem, acc_sh):
        cid = lax.axis_index("c")
        sid = lax.axis_index("s")
        wid = sid * NC + cid

        def zfill(i, _):
            for g in range(d // L):
                rows_v[i, pl.ds(g * L, L)] = jnp.zeros((L,), jnp.float32)
            return ()
        lax.fori_loop(0, CHUNK, zfill, (), unroll=False)

        def zbody(i, _):
            pltpu.sync_copy(
                rows_v, acc_sh.at[pl.ds(sid * npt + i * CHUNK, CHUNK)])
            return ()
        lax.fori_loop(0, npt // CHUNK, zbody, (), unroll=False)
        plsc.subcore_barrier()

        pltpu.sync_copy(row_hbm.at[wid], row_v)
        pltpu.sync_copy(col_hbm.at[wid], col_v)
        pltpu.sync_copy(w_hbm.at[wid], w_v)

        def body(j, _):
            pltpu.async_copy(x_hbm.at[row_v.at[j]], rows_v, sem).wait()

            def rbody(g, _):
                base = g * L
                for i2 in range(L):
                    r = base + i2
                    wb = plsc.load_gather(w_v, [_full16(j), _full16(r)])
                    for dd in range(d // L):
                        sl = pl.ds(dd * L, L)
                        rows_v[r, sl] = rows_v[r, sl] * wb
                return ()
            lax.fori_loop(0, CHUNK // L, rbody, (), unroll=False)

            pltpu.sync_copy(rows_v, acc_sh.at[col_v.at[j]], add=True)
            return ()
        lax.fori_loop(0, nchunks, body, (), unroll=False)

        plsc.subcore_barrier()
        pltpu.sync_copy(acc_sh.at[pl.ds(sid * npt, npt)],
                        part_hbm.at[cid, pl.ds(sid * npt, npt)])

    hop = pl.kernel(
        _hop_body,
        out_type=jax.ShapeDtypeStruct((NC, n_pad, d), jnp.float32),
        mesh=plsc.VectorSubcoreMesh(core_axis_name="c", subcore_axis_name="s", num_cores=NC, num_subcores=NS),
        compiler_params=pltpu.CompilerParams(needs_layout_passes=False),
        scratch_types=[
            pltpu.VMEM((nchunks, CHUNK), jnp.int32),
            pltpu.VMEM((nchunks, CHUNK), jnp.int32),
            pltpu.VMEM((nchunks, CHUNK), jnp.float32),
            pltpu.VMEM((CHUNK, d), jnp.float32),
            pltpu.SemaphoreType.DMA,
            pltpu.VMEM_SHARED((n_pad, d), jnp.float32),
        ],
    )

    combine = pl.pallas_call(
        functools.partial(_combine_body, n),
        out_shape=jax.ShapeDtypeStruct((n, d), jnp.float32),
    )

    xs = [x]
    for _ in range(K_HOPS):
        part = hop(xs[-1], row_p, col_p, w)
        xs.append(combine(part.reshape(2 * n_pad, d)))
    return jnp.stack(xs, axis=0)
